# Initial kernel scaffold; baseline (speedup 1.0000x reference)
import jax
import jax.numpy as jnp
from jax.experimental import pallas as pl


def kernel(x, edge_index, W1, b1, W2, b2, W3, b3):
    raise NotImplementedError("write your pallas kernel here")


# speedup vs baseline: 2.8670x; 2.8670x over previous
import functools

import jax
import jax.numpy as jnp
from jax import lax
from jax.experimental import pallas as pl
from jax.experimental.pallas import tpu as pltpu
from jax.experimental.pallas import tpu_sc as plsc

D = 128
NC, NS = 2, 16
NW = NC * NS
CHUNK = 128
ROW_CHUNKS = 79
N_PAD = ROW_CHUNKS * CHUNK
PT = 80
E_PAD = NW * PT * CHUNK
IDXB = 40
NBLK = PT // IDXB

_MESH = plsc.VectorSubcoreMesh(
    core_axis_name="c", subcore_axis_name="s", num_cores=NC, num_subcores=NS)


_ZK = (ROW_CHUNKS + NS - 1) // NS


@functools.partial(
    pl.kernel,
    out_type=jax.ShapeDtypeStruct((2, NC, N_PAD, D), jnp.float32),
    mesh=_MESH,
    scratch_types=[
        pltpu.VMEM((CHUNK,), jnp.int32),
        pltpu.VMEM((CHUNK, D), jnp.float32),
        pltpu.VMEM_SHARED((N_PAD, D), jnp.float32),
    ],
)
def _deg_pass(src_hbm, dst_hbm, ones_hbm, z_hbm, out_hbm, idxv, ones_v, acc):
    cid = lax.axis_index("c")
    sid = lax.axis_index("s")
    wid = sid * NC + cid
    pltpu.sync_copy(ones_hbm, ones_v)
    for phase, idx_hbm in enumerate((src_hbm, dst_hbm)):
        for k in range(_ZK):
            c = sid + NS * k
            @pl.when(c < ROW_CHUNKS)
            def _():
                pltpu.sync_copy(z_hbm, acc.at[pl.ds(c * CHUNK, CHUNK)])
        plsc.subcore_barrier()

        def body(i, _):
            base = (wid * PT + i) * CHUNK
            pltpu.sync_copy(idx_hbm.at[pl.ds(base, CHUNK)], idxv)
            pltpu.sync_copy(ones_v, acc.at[idxv], add=True)
            return ()
        lax.fori_loop(0, PT, body, ())

        plsc.subcore_barrier()
        for k in range(_ZK):
            c = sid + NS * k
            @pl.when(c < ROW_CHUNKS)
            def _():
                rows = pl.ds(c * CHUNK, CHUNK)
                pltpu.sync_copy(acc.at[rows], out_hbm.at[phase, cid, rows])
        plsc.subcore_barrier()


@functools.partial(
    pl.kernel,
    out_type=jax.ShapeDtypeStruct((NC, N_PAD, D), jnp.float32),
    mesh=_MESH,
    scratch_types=[
        pltpu.VMEM((IDXB, CHUNK), jnp.int32),
        pltpu.VMEM((CHUNK,), jnp.int32),
        pltpu.VMEM((CHUNK,), jnp.int32),
        pltpu.VMEM((CHUNK, D), jnp.float32),
        pltpu.VMEM((CHUNK, D), jnp.float32),
        pltpu.VMEM_SHARED((N_PAD, D), jnp.float32),
        pltpu.SemaphoreType.DMA,
        pltpu.SemaphoreType.DMA,
    ],
)
def _edge_pass(h_hbm, src_hbm, dst_hbm, z_hbm, out_hbm,
               sidx, didx_a, didx_b, rows_a, rows_b, acc, sem_a, sem_b):
    cid = lax.axis_index("c")
    sid = lax.axis_index("s")
    wid = sid * NC + cid
    for k in range(_ZK):
        c = sid + NS * k
        @pl.when(c < ROW_CHUNKS)
        def _():
            pltpu.sync_copy(z_hbm, acc.at[pl.ds(c * CHUNK, CHUNK)])
    plsc.subcore_barrier()

    for blk in range(NBLK):
        cbase = wid * PT + blk * IDXB
        pltpu.sync_copy(src_hbm.at[pl.ds(cbase, IDXB)], sidx)
        pltpu.async_copy(h_hbm.at[sidx.at[0]], rows_a, sem_a)

        def body(j, _):
            i0 = 2 * j
            pltpu.async_copy(h_hbm.at[sidx.at[i0 + 1]], rows_b, sem_b)
            pltpu.sync_copy(dst_hbm.at[pl.ds((cbase + i0) * CHUNK, CHUNK)],
                            didx_a)
            pltpu.make_async_copy(h_hbm.at[sidx.at[i0]], rows_a, sem_a).wait()
            pltpu.sync_copy(rows_a, acc.at[didx_a], add=True)
            @pl.when(i0 + 2 < IDXB)
            def _():
                pltpu.async_copy(h_hbm.at[sidx.at[i0 + 2]], rows_a, sem_a)
            pltpu.sync_copy(dst_hbm.at[pl.ds((cbase + i0 + 1) * CHUNK, CHUNK)],
                            didx_b)
            pltpu.make_async_copy(h_hbm.at[sidx.at[i0 + 1]], rows_b, sem_b).wait()
            pltpu.sync_copy(rows_b, acc.at[didx_b], add=True)
            return ()
        lax.fori_loop(0, IDXB // 2, body, ())

    plsc.subcore_barrier()
    for k in range(_ZK):
        c = sid + NS * k
        @pl.when(c < ROW_CHUNKS)
        def _():
            rows = pl.ds(c * CHUNK, CHUNK)
            pltpu.sync_copy(acc.at[rows], out_hbm.at[cid, rows])


def _norm(d0, d1):
    return lax.rsqrt(jnp.clip(d0 + d1, 1.0, None))


def _scale_body(x_ref, ds0, ds1, o_ref):
    o_ref[...] = x_ref[...] * _norm(ds0[...], ds1[...])


def _post_elu_body(p0, p1, dd0, dd1, ds0, ds1, w, b, hact, hsc):
    rst = (p0[...] + p1[...]) * _norm(dd0[...], dd1[...])
    y = jnp.dot(rst, w[...], preferred_element_type=jnp.float32) + b[...]
    a = jnp.where(y > 0, y, jnp.exp(jnp.minimum(y, 0.0)) - 1.0)
    hact[...] = a
    hsc[...] = a * _norm(ds0[...], ds1[...])


def _post_softmax_body(p0, p1, dd0, dd1, w, b, out):
    rst = (p0[...] + p1[...]) * _norm(dd0[...], dd1[...])
    y = jnp.dot(rst, w[...], preferred_element_type=jnp.float32) + b[...]
    m = jnp.max(y, axis=1, keepdims=True)
    e = jnp.exp(y - m)
    out[...] = e / jnp.sum(e, axis=1, keepdims=True)


_row_blk = pl.BlockSpec((CHUNK, D), lambda i: (i, 0))
_vec_blk = pl.BlockSpec((CHUNK, 1), lambda i: (i, 0))
_w_blk = pl.BlockSpec((D, D), lambda i: (0, 0))
_b_blk = pl.BlockSpec((1, D), lambda i: (0, 0))

_scale = pl.pallas_call(
    _scale_body,
    grid=(ROW_CHUNKS,),
    in_specs=[_row_blk, _vec_blk, _vec_blk],
    out_specs=_row_blk,
    out_shape=jax.ShapeDtypeStruct((N_PAD, D), jnp.float32),
)

_post_elu = pl.pallas_call(
    _post_elu_body,
    grid=(ROW_CHUNKS,),
    in_specs=[_row_blk, _row_blk, _vec_blk, _vec_blk, _vec_blk, _vec_blk,
              _w_blk, _b_blk],
    out_specs=[_row_blk, _row_blk],
    out_shape=[jax.ShapeDtypeStruct((N_PAD, D), jnp.float32),
               jax.ShapeDtypeStruct((N_PAD, D), jnp.float32)],
)

_post_softmax = pl.pallas_call(
    _post_softmax_body,
    grid=(ROW_CHUNKS,),
    in_specs=[_row_blk, _row_blk, _vec_blk, _vec_blk, _w_blk, _b_blk],
    out_specs=_row_blk,
    out_shape=jax.ShapeDtypeStruct((N_PAD, D), jnp.float32),
)


def kernel(x, edge_index, W1, b1, W2, b2, W3, b3):
    n = x.shape[0]
    e = edge_index.shape[1]
    src = edge_index[0].astype(jnp.int32)
    dst = edge_index[1].astype(jnp.int32)
    pad = jnp.full((E_PAD - e,), N_PAD - 1, jnp.int32)
    src_f = jnp.concatenate([src, pad])
    dst_f = jnp.concatenate([dst, pad])
    src2d = src_f.reshape(NW * PT, CHUNK)

    ones_r = jnp.ones((CHUNK, D), jnp.float32)
    zrow = jnp.zeros((CHUNK, D), jnp.float32)

    degs = _deg_pass(src_f, dst_f, ones_r, zrow)
    ds0 = degs[0, 0, :, 0].reshape(N_PAD, 1)
    ds1 = degs[0, 1, :, 0].reshape(N_PAD, 1)
    dd0 = degs[1, 0, :, 0].reshape(N_PAD, 1)
    dd1 = degs[1, 1, :, 0].reshape(N_PAD, 1)

    x_pad = jnp.pad(x, ((0, N_PAD - n), (0, 0)))
    b1r = b1.reshape(1, D)
    b2r = b2.reshape(1, D)
    b3r = b3.reshape(1, D)

    h = _scale(x_pad, ds0, ds1)
    parts = _edge_pass(h, src2d, dst_f, zrow)
    h1, h = _post_elu(parts[0], parts[1], dd0, dd1, ds0, ds1, W1, b1r)
    parts = _edge_pass(h, src2d, dst_f, zrow)
    h2, h = _post_elu(parts[0], parts[1], dd0, dd1, ds0, ds1, W2, b2r)
    parts = _edge_pass(h, src2d, dst_f, zrow)
    h3 = _post_softmax(parts[0], parts[1], dd0, dd1, W3, b3r)

    d = x.shape[1]
    return (h1[:n].reshape(-1, n, d), h2[:n].reshape(-1, n, d),
            h3[:n].reshape(-1, n, d))


# speedup vs baseline: 6.4126x; 2.2367x over previous
import functools

import jax
import jax.numpy as jnp
from jax import lax
from jax.experimental import pallas as pl
from jax.experimental.pallas import tpu as pltpu
from jax.experimental.pallas import tpu_sc as plsc

D = 128
NC, NS = 2, 16
NW = NC * NS
CHUNK = 128
ROW_CHUNKS = 79
N_PAD = ROW_CHUNKS * CHUNK
PT = 80
E_PAD = NW * PT * CHUNK
IDXB = 40
NBLK = PT // IDXB

_MESH = plsc.VectorSubcoreMesh(
    core_axis_name="c", subcore_axis_name="s", num_cores=NC, num_subcores=NS)


_ZK = (ROW_CHUNKS + NS - 1) // NS


@functools.partial(
    pl.kernel,
    out_type=jax.ShapeDtypeStruct((2, NC, N_PAD, D), jnp.float32),
    mesh=_MESH,
    scratch_types=[
        pltpu.VMEM((CHUNK,), jnp.int32),
        pltpu.VMEM((CHUNK, D), jnp.float32),
        pltpu.VMEM_SHARED((N_PAD, D), jnp.float32),
    ],
)
def _deg_pass(src_hbm, dst_hbm, ones_hbm, z_hbm, out_hbm, idxv, ones_v, acc):
    cid = lax.axis_index("c")
    sid = lax.axis_index("s")
    wid = sid * NC + cid
    pltpu.sync_copy(ones_hbm, ones_v)
    for phase, idx_hbm in enumerate((src_hbm, dst_hbm)):
        for k in range(_ZK):
            c = sid + NS * k
            @pl.when(c < ROW_CHUNKS)
            def _():
                pltpu.sync_copy(z_hbm, acc.at[pl.ds(c * CHUNK, CHUNK)])
        plsc.subcore_barrier()

        def body(i, _):
            base = (wid * PT + i) * CHUNK
            pltpu.sync_copy(idx_hbm.at[pl.ds(base, CHUNK)], idxv)
            pltpu.sync_copy(ones_v, acc.at[idxv], add=True)
            return ()
        lax.fori_loop(0, PT, body, ())

        plsc.subcore_barrier()
        for k in range(_ZK):
            c = sid + NS * k
            @pl.when(c < ROW_CHUNKS)
            def _():
                rows = pl.ds(c * CHUNK, CHUNK)
                pltpu.sync_copy(acc.at[rows], out_hbm.at[phase, cid, rows])
        plsc.subcore_barrier()


@functools.partial(
    pl.kernel,
    out_type=jax.ShapeDtypeStruct((NC, N_PAD, D), jnp.float32),
    mesh=_MESH,
    scratch_types=[
        pltpu.VMEM((IDXB, CHUNK), jnp.int32),
        pltpu.VMEM((CHUNK,), jnp.int32),
        pltpu.VMEM((CHUNK,), jnp.int32),
        pltpu.VMEM((CHUNK, D), jnp.float32),
        pltpu.VMEM((CHUNK, D), jnp.float32),
        pltpu.VMEM_SHARED((N_PAD, D), jnp.float32),
        pltpu.SemaphoreType.DMA,
        pltpu.SemaphoreType.DMA,
    ],
)
def _edge_pass(h_hbm, src_hbm, dst_hbm, z_hbm, out_hbm,
               sidx, didx_a, didx_b, rows_a, rows_b, acc, sem_a, sem_b):
    cid = lax.axis_index("c")
    sid = lax.axis_index("s")
    wid = sid * NC + cid
    for k in range(_ZK):
        c = sid + NS * k
        @pl.when(c < ROW_CHUNKS)
        def _():
            pltpu.sync_copy(z_hbm, acc.at[pl.ds(c * CHUNK, CHUNK)])
    plsc.subcore_barrier()

    for blk in range(NBLK):
        cbase = wid * PT + blk * IDXB
        pltpu.sync_copy(src_hbm.at[pl.ds(cbase, IDXB)], sidx)
        pltpu.async_copy(h_hbm.at[sidx.at[0]], rows_a, sem_a)

        def body(j, _):
            i0 = 2 * j
            pltpu.async_copy(h_hbm.at[sidx.at[i0 + 1]], rows_b, sem_b)
            pltpu.sync_copy(dst_hbm.at[pl.ds((cbase + i0) * CHUNK, CHUNK)],
                            didx_a)
            pltpu.make_async_copy(h_hbm.at[sidx.at[i0]], rows_a, sem_a).wait()
            pltpu.sync_copy(rows_a, acc.at[didx_a], add=True)
            @pl.when(i0 + 2 < IDXB)
            def _():
                pltpu.async_copy(h_hbm.at[sidx.at[i0 + 2]], rows_a, sem_a)
            pltpu.sync_copy(dst_hbm.at[pl.ds((cbase + i0 + 1) * CHUNK, CHUNK)],
                            didx_b)
            pltpu.make_async_copy(h_hbm.at[sidx.at[i0 + 1]], rows_b, sem_b).wait()
            pltpu.sync_copy(rows_b, acc.at[didx_b], add=True)
            return ()
        lax.fori_loop(0, IDXB // 2, body, ())

    plsc.subcore_barrier()
    for k in range(_ZK):
        c = sid + NS * k
        @pl.when(c < ROW_CHUNKS)
        def _():
            rows = pl.ds(c * CHUNK, CHUNK)
            pltpu.sync_copy(acc.at[rows], out_hbm.at[cid, rows])


def _norm(d0, d1):
    return lax.rsqrt(jnp.clip(d0 + d1, 1.0, None))


def _scale_body(x_ref, ds0, ds1, o_ref):
    o_ref[...] = x_ref[...] * _norm(ds0[...], ds1[...])


def _post_elu_body(p0, p1, dd0, dd1, ds0, ds1, w, b, hact, hsc):
    rst = (p0[...] + p1[...]) * _norm(dd0[...], dd1[...])
    y = jnp.dot(rst, w[...], preferred_element_type=jnp.float32) + b[...]
    a = jnp.where(y > 0, y, jnp.exp(jnp.minimum(y, 0.0)) - 1.0)
    hact[...] = a
    hsc[...] = a * _norm(ds0[...], ds1[...])


def _post_softmax_body(p0, p1, dd0, dd1, w, b, out):
    rst = (p0[...] + p1[...]) * _norm(dd0[...], dd1[...])
    y = jnp.dot(rst, w[...], preferred_element_type=jnp.float32) + b[...]
    m = jnp.max(y, axis=1, keepdims=True)
    e = jnp.exp(y - m)
    out[...] = e / jnp.sum(e, axis=1, keepdims=True)


_row_blk = pl.BlockSpec((CHUNK, D), lambda i: (i, 0))
_vec_blk = pl.BlockSpec((CHUNK, 1), lambda i: (i, 0))
_w_blk = pl.BlockSpec((D, D), lambda i: (0, 0))
_b_blk = pl.BlockSpec((1, D), lambda i: (0, 0))

_scale = pl.pallas_call(
    _scale_body,
    grid=(ROW_CHUNKS,),
    in_specs=[_row_blk, _vec_blk, _vec_blk],
    out_specs=_row_blk,
    out_shape=jax.ShapeDtypeStruct((N_PAD, D), jnp.float32),
)

_post_elu = pl.pallas_call(
    _post_elu_body,
    grid=(ROW_CHUNKS,),
    in_specs=[_row_blk, _row_blk, _vec_blk, _vec_blk, _vec_blk, _vec_blk,
              _w_blk, _b_blk],
    out_specs=[_row_blk, _row_blk],
    out_shape=[jax.ShapeDtypeStruct((N_PAD, D), jnp.float32),
               jax.ShapeDtypeStruct((N_PAD, D), jnp.float32)],
)

_post_softmax = pl.pallas_call(
    _post_softmax_body,
    grid=(ROW_CHUNKS,),
    in_specs=[_row_blk, _row_blk, _vec_blk, _vec_blk, _w_blk, _b_blk],
    out_specs=_row_blk,
    out_shape=jax.ShapeDtypeStruct((N_PAD, D), jnp.float32),
)


def kernel(x, edge_index, W1, b1, W2, b2, W3, b3):
    n = x.shape[0]
    e = edge_index.shape[1]
    src = edge_index[0].astype(jnp.int32)
    dst = edge_index[1].astype(jnp.int32)
    pad = n + (jnp.arange(E_PAD - e, dtype=jnp.int32) % (N_PAD - n))
    src_f = jnp.concatenate([src, pad])
    dst_f = jnp.concatenate([dst, pad])
    src2d = src_f.reshape(NW * PT, CHUNK)

    ones_r = jnp.ones((CHUNK, D), jnp.float32)
    zrow = jnp.zeros((CHUNK, D), jnp.float32)

    degs = _deg_pass(src_f, dst_f, ones_r, zrow)
    ds0 = degs[0, 0, :, 0].reshape(N_PAD, 1)
    ds1 = degs[0, 1, :, 0].reshape(N_PAD, 1)
    dd0 = degs[1, 0, :, 0].reshape(N_PAD, 1)
    dd1 = degs[1, 1, :, 0].reshape(N_PAD, 1)

    x_pad = jnp.pad(x, ((0, N_PAD - n), (0, 0)))
    b1r = b1.reshape(1, D)
    b2r = b2.reshape(1, D)
    b3r = b3.reshape(1, D)

    h = _scale(x_pad, ds0, ds1)
    parts = _edge_pass(h, src2d, dst_f, zrow)
    h1, h = _post_elu(parts[0], parts[1], dd0, dd1, ds0, ds1, W1, b1r)
    parts = _edge_pass(h, src2d, dst_f, zrow)
    h2, h = _post_elu(parts[0], parts[1], dd0, dd1, ds0, ds1, W2, b2r)
    parts = _edge_pass(h, src2d, dst_f, zrow)
    h3 = _post_softmax(parts[0], parts[1], dd0, dd1, W3, b3r)

    d = x.shape[1]
    return (h1[:n].reshape(-1, n, d), h2[:n].reshape(-1, n, d),
            h3[:n].reshape(-1, n, d))
